# baseline (speedup 1.0000x reference)
import functools

import jax
import jax.numpy as jnp
from jax.experimental import pallas as pl
from jax.experimental.pallas import tpu as pltpu

_B, _N, _T, _K, _D, _V, _TOK = 8, 256, 8, 7, 256, 7, 64
_KD = _K * _D
_VP = 8
_NPAIR = _B * _N
_PB = 512
_PBV = 512


def _vis_kernel(vis_ref, m_ref, wvis_ref, bias_ref, out_ref):
    m = m_ref[...]
    s = jnp.sum(m, axis=1, keepdims=True)
    scale = jnp.where(s > 0.0, 1.0 / jnp.maximum(s, 1.0), 0.0)
    w = m * scale

    vis = vis_ref[...]
    vw = jnp.sum(vis * w[:, :, None], axis=1)
    out = jax.lax.dot_general(vw, wvis_ref[...], (((1,), (0,)), ((), ())),
                              preferred_element_type=jnp.float32)
    any_m = (s > 0.0).astype(jnp.float32)
    out_ref[...] = out + any_m * bias_ref[...]


def _emb_kernel(emb_ref, m_ref, wemb_ref, vb_ref, out_ref):
    m = m_ref[...]
    s = jnp.sum(m, axis=1, keepdims=True)
    scale = jnp.where(s > 0.0, 1.0 / jnp.maximum(s, 1.0), 0.0)
    w = m * scale

    e = emb_ref[...]
    ew = jnp.sum(e * w[:, :, None], axis=1)
    out = jax.lax.dot_general(ew, wemb_ref[...], (((1,), (0,)), ((), ())),
                              preferred_element_type=jnp.float32)
    out_ref[...] = out + vb_ref[...]


@jax.jit
def kernel(embeddings, visibility_scores, masks, W, b):
    emb = embeddings.reshape(_NPAIR, _T, _KD)
    vis = jnp.pad(visibility_scores, ((0, 0), (0, 0), (0, 0), (0, _VP - _V)))
    vis = vis.reshape(_NPAIR, _T, _VP)
    m = masks.astype(jnp.float32).reshape(_NPAIR, _T)
    wemb = W[:, :_KD].T
    wvis = jnp.pad(W[:, _KD:], ((0, 0), (0, _VP - _V))).T
    bias = b.reshape(1, _TOK)

    vb = pl.pallas_call(
        _vis_kernel,
        grid=(_NPAIR // _PBV,),
        in_specs=[
            pl.BlockSpec((_PBV, _T, _VP), lambda j: (j, 0, 0)),
            pl.BlockSpec((_PBV, _T), lambda j: (j, 0)),
            pl.BlockSpec((_VP, _TOK), lambda j: (0, 0)),
            pl.BlockSpec((1, _TOK), lambda j: (0, 0)),
        ],
        out_specs=pl.BlockSpec((_PBV, _TOK), lambda j: (j, 0)),
        out_shape=jax.ShapeDtypeStruct((_NPAIR, _TOK), jnp.float32),
    )(vis, m, wvis, bias)

    out = pl.pallas_call(
        _emb_kernel,
        grid=(_NPAIR // _PB,),
        in_specs=[
            pl.BlockSpec((_PB, _T, _KD), lambda j: (j, 0, 0)),
            pl.BlockSpec((_PB, _T), lambda j: (j, 0)),
            pl.BlockSpec((_KD, _TOK), lambda j: (0, 0)),
            pl.BlockSpec((_PB, _TOK), lambda j: (j, 0)),
        ],
        out_specs=pl.BlockSpec((_PB, _TOK), lambda j: (j, 0)),
        out_shape=jax.ShapeDtypeStruct((_NPAIR, _TOK), jnp.float32),
        compiler_params=pltpu.CompilerParams(
            vmem_limit_bytes=100 * 1024 * 1024),
    )(emb, m, wemb, vb)
    return out.reshape(_B, _N, _TOK)


# speedup vs baseline: 1.1092x; 1.1092x over previous
import functools

import jax
import jax.numpy as jnp
from jax.experimental import pallas as pl
from jax.experimental.pallas import tpu as pltpu

_B, _N, _T, _K, _D, _V, _TOK = 8, 256, 8, 7, 256, 7, 64
_KD = _K * _D
_VP = 8
_NB = 256


def _proj_kernel(emb_ref, vis_ref, m_ref, wemb_ref, wvis_ref, bias_ref, out_ref):
    m = m_ref[0]
    s = jnp.sum(m, axis=1, keepdims=True)
    scale = jnp.where(s > 0.0, 1.0 / jnp.maximum(s, 1.0), 0.0)
    w = m * scale

    e = emb_ref[0]
    ew = jnp.sum(e * w[:, :, None], axis=1)
    vis = vis_ref[0]
    vw = jnp.sum(vis * w[:, :, None], axis=1)

    acc = jax.lax.dot_general(ew, wemb_ref[...], (((1,), (0,)), ((), ())),
                              preferred_element_type=jnp.float32)
    acc = acc + jax.lax.dot_general(vw, wvis_ref[...], (((1,), (0,)), ((), ())),
                                    preferred_element_type=jnp.float32)
    any_m = (s > 0.0).astype(jnp.float32)
    out_ref[0] = acc + any_m * bias_ref[...]


@jax.jit
def kernel(embeddings, visibility_scores, masks, W, b):
    emb = embeddings.reshape(_B, _N, _T, _KD)
    vis = jnp.pad(visibility_scores, ((0, 0), (0, 0), (0, 0), (0, _VP - _V)))
    m = masks.astype(jnp.float32)
    wemb = W[:, :_KD].T
    wvis = jnp.pad(W[:, _KD:], ((0, 0), (0, _VP - _V))).T
    bias = b.reshape(1, _TOK)

    grid = (_B, _N // _NB)
    return pl.pallas_call(
        _proj_kernel,
        grid=grid,
        in_specs=[
            pl.BlockSpec((1, _NB, _T, _KD), lambda i, j: (i, j, 0, 0)),
            pl.BlockSpec((1, _NB, _T, _VP), lambda i, j: (i, j, 0, 0)),
            pl.BlockSpec((1, _NB, _T), lambda i, j: (i, j, 0)),
            pl.BlockSpec((_KD, _TOK), lambda i, j: (0, 0)),
            pl.BlockSpec((_VP, _TOK), lambda i, j: (0, 0)),
            pl.BlockSpec((1, _TOK), lambda i, j: (0, 0)),
        ],
        out_specs=pl.BlockSpec((1, _NB, _TOK), lambda i, j: (i, j, 0)),
        out_shape=jax.ShapeDtypeStruct((_B, _N, _TOK), jnp.float32),
    )(emb, vis, m, wemb, wvis, bias)
